# Initial kernel scaffold; baseline (speedup 1.0000x reference)
import jax
import jax.numpy as jnp
from jax.experimental import pallas as pl


def kernel(x, edge_index, edge_weight, W1, b1, W2, b2):
    raise NotImplementedError("write your pallas kernel here")


# speedup vs baseline: 34.6592x; 34.6592x over previous
import functools

import jax
import jax.numpy as jnp
from jax import lax
from jax.experimental import pallas as pl
from jax.experimental.pallas import tpu as pltpu
from jax.experimental.pallas import tpu_sc as plsc

N = 10000
E = 320000
D_IN = 128
D_H = 16
D_OUT = 40

NC, NS = 2, 16
NW = NC * NS
BLK = 128
BPT = 80
EP = NW * BPT * BLK
CHUNK = 16
NPAD = 10240
HR = NPAD // 16
STRIPE = NPAD // NS


def _sc_deg_body(col_hbm, ew_hbm, out_hbm, colv, ewv, hist):
  cid = lax.axis_index("c")
  sid = lax.axis_index("s")
  wid = cid * NS + sid
  pltpu.sync_copy(col_hbm.at[pl.ds(wid * BPT, BPT)], colv)
  pltpu.sync_copy(ew_hbm.at[pl.ds(wid * BPT, BPT)], ewv)

  zero = jnp.zeros((16,), jnp.float32)

  def z(i, carry):
    hist[pl.ds(i * 16, 16)] = zero
    return carry

  lax.fori_loop(0, HR, z, 0)

  def grp(i, carry):
    b = i >> 3
    g = (i & 7) * 16
    idx = colv[b, pl.ds(g, 16)]
    w = ewv[b, pl.ds(g, 16)]
    plsc.addupdate_scatter(hist, [idx], w)
    return carry

  lax.fori_loop(0, BPT * 8, grp, 0)
  pltpu.sync_copy(hist, out_hbm.at[wid])


def _sc_agg_body(g_hbm, row_hbm, col_hbm, ew_hbm, out_hbm,
                 rowv, colv, ewv, rows, acc, sem):
  cid = lax.axis_index("c")
  sid = lax.axis_index("s")
  wid = cid * NS + sid
  pltpu.sync_copy(row_hbm.at[pl.ds(wid * BPT, BPT)], rowv)
  pltpu.sync_copy(col_hbm.at[pl.ds(wid * BPT, BPT)], colv)
  pltpu.sync_copy(ew_hbm.at[pl.ds(wid * BPT, BPT)], ewv)

  zero = jnp.zeros((16,), jnp.float32)

  def z(i, carry):
    rows[i, :] = zero
    return carry

  lax.fori_loop(0, STRIPE, z, 0)
  pltpu.sync_copy(rows.at[pl.ds(0, STRIPE)], acc.at[pl.ds(sid * STRIPE, STRIPE)])
  plsc.subcore_barrier()

  for c in range(BPT // CHUNK):
    descs = []
    for b in range(CHUNK):
      d = pltpu.async_copy(g_hbm.at[rowv.at[c * CHUNK + b]],
                           rows.at[pl.ds(b * BLK, BLK)], sem)
      descs.append(d)
    for d in descs:
      d.wait()

    def scale(g, carry):
      blk = g >> 3
      off = (g & 7) * 16
      w16 = ewv[c * CHUNK + blk, pl.ds(off, 16)]
      base = blk * BLK + off
      for l in range(16):
        rows[base + l, :] = rows[base + l, :] * w16[l]
      return carry

    lax.fori_loop(0, CHUNK * 8, scale, 0)

    for b in range(CHUNK):
      pltpu.sync_copy(rows.at[pl.ds(b * BLK, BLK)],
                      acc.at[colv.at[c * CHUNK + b]], add=True)

  plsc.subcore_barrier()
  pltpu.sync_copy(acc.at[pl.ds(sid * STRIPE, STRIPE)],
                  out_hbm.at[cid, pl.ds(sid * STRIPE, STRIPE)])


def _sc_calls():
  mesh = plsc.VectorSubcoreMesh(
      core_axis_name="c", subcore_axis_name="s",
      num_cores=NC, num_subcores=NS)
  deg = pl.kernel(
      _sc_deg_body,
      out_type=jax.ShapeDtypeStruct((NW, NPAD), jnp.float32),
      mesh=mesh,
      scratch_types=[
          pltpu.VMEM((BPT, BLK), jnp.int32),
          pltpu.VMEM((BPT, BLK), jnp.float32),
          pltpu.VMEM((NPAD,), jnp.float32),
      ],
      compiler_params=pltpu.CompilerParams(needs_layout_passes=False),
  )
  agg = pl.kernel(
      _sc_agg_body,
      out_type=jax.ShapeDtypeStruct((NC, NPAD, 16), jnp.float32),
      mesh=mesh,
      scratch_types=[
          pltpu.VMEM((BPT, BLK), jnp.int32),
          pltpu.VMEM((BPT, BLK), jnp.int32),
          pltpu.VMEM((BPT, BLK), jnp.float32),
          pltpu.VMEM((CHUNK * BLK, 16), jnp.float32),
          pltpu.VMEM_SHARED((NPAD, 16), jnp.float32),
          pltpu.SemaphoreType.DMA,
      ],
      compiler_params=pltpu.CompilerParams(
          needs_layout_passes=False, use_tc_tiling_on_sc=False),
  )
  return deg, agg


RB = 2000


def _mm1_body(x_ref, w_ref, o_ref):
  o_ref[...] = jnp.dot(x_ref[...], w_ref[...],
                       preferred_element_type=jnp.float32)


def _dis_body(hist_ref, o_ref):
  deg = jnp.sum(hist_ref[...], axis=0) + 1.0
  o_ref[...] = lax.rsqrt(deg)


def _scale_body(h_ref, d_ref, g_ref, s_ref):
  d = d_ref[...]
  h = h_ref[...]
  g_ref[...] = h * d
  s_ref[...] = h * (d * d)


def _mid_body(p_ref, s_ref, d_ref, b_ref, g_ref, so_ref):
  d = d_ref[...]
  o1 = jnp.maximum((p_ref[0] + p_ref[1]) * d + s_ref[...] + b_ref[...], 0.0)
  g_ref[...] = o1 * d
  so_ref[...] = o1 * (d * d)


def _fin_body(p_ref, so_ref, d_ref, w_ref, b_ref, o_ref):
  agg = (p_ref[0] + p_ref[1]) * d_ref[...] + so_ref[...]
  z = jnp.dot(agg, w_ref[...], preferred_element_type=jnp.float32) + b_ref[...]
  m = jnp.max(z, axis=1, keepdims=True)
  zz = z - m
  lse = jnp.log(jnp.sum(jnp.exp(zz), axis=1, keepdims=True))
  o_ref[...] = zz - lse


def _row_spec(width):
  return pl.BlockSpec((RB, width), lambda i: (i, 0))


def _part_spec():
  return pl.BlockSpec((NC, RB, 16), lambda i: (0, i, 0))


def _full_spec(shape):
  nd = len(shape)
  return pl.BlockSpec(shape, lambda i: (0,) * nd)


def kernel(x, edge_index, edge_weight, W1, b1, W2, b2):
  row = edge_index[0].astype(jnp.int32)
  col = edge_index[1].astype(jnp.int32)
  ew = edge_weight.astype(jnp.float32)
  pad = EP - E
  row2d = jnp.concatenate([row, jnp.zeros((pad,), jnp.int32)]).reshape(NW * BPT, BLK)
  col2d = jnp.concatenate([col, jnp.zeros((pad,), jnp.int32)]).reshape(NW * BPT, BLK)
  ew2d = jnp.concatenate([ew, jnp.zeros((pad,), jnp.float32)]).reshape(NW * BPT, BLK)

  deg_call, agg_call = _sc_calls()

  h = pl.pallas_call(
      _mm1_body,
      grid=(N // RB,),
      in_specs=[pl.BlockSpec((RB, D_IN), lambda i: (i, 0)),
                _full_spec((D_IN, D_H))],
      out_specs=_row_spec(D_H),
      out_shape=jax.ShapeDtypeStruct((N, D_H), jnp.float32),
  )(x, W1)

  hists = deg_call(col2d, ew2d)

  dis_flat = pl.pallas_call(
      _dis_body,
      out_shape=jax.ShapeDtypeStruct((NPAD,), jnp.float32),
  )(hists)
  dis_col = dis_flat.reshape(NPAD, 1)[:N]

  g1, selfh = pl.pallas_call(
      _scale_body,
      grid=(N // RB,),
      in_specs=[_row_spec(D_H), _row_spec(1)],
      out_specs=[_row_spec(D_H), _row_spec(D_H)],
      out_shape=[jax.ShapeDtypeStruct((N, D_H), jnp.float32),
                 jax.ShapeDtypeStruct((N, D_H), jnp.float32)],
  )(h, dis_col)

  parts1 = agg_call(g1, row2d, col2d, ew2d)

  g2, selfo = pl.pallas_call(
      _mid_body,
      grid=(N // RB,),
      in_specs=[_part_spec(), _row_spec(D_H), _row_spec(1),
                _full_spec((1, D_H))],
      out_specs=[_row_spec(D_H), _row_spec(D_H)],
      out_shape=[jax.ShapeDtypeStruct((N, D_H), jnp.float32),
                 jax.ShapeDtypeStruct((N, D_H), jnp.float32)],
  )(parts1, selfh, dis_col, b1.reshape(1, D_H))

  parts2 = agg_call(g2, row2d, col2d, ew2d)

  out = pl.pallas_call(
      _fin_body,
      grid=(N // RB,),
      in_specs=[_part_spec(), _row_spec(D_H), _row_spec(1),
                _full_spec((D_H, D_OUT)), _full_spec((1, D_OUT))],
      out_specs=_row_spec(D_OUT),
      out_shape=jax.ShapeDtypeStruct((N, D_OUT), jnp.float32),
  )(parts2, selfo, dis_col, W2, b2.reshape(1, D_OUT))

  return out


# speedup vs baseline: 38.8236x; 1.1202x over previous
import functools

import jax
import jax.numpy as jnp
from jax import lax
from jax.experimental import pallas as pl
from jax.experimental.pallas import tpu as pltpu
from jax.experimental.pallas import tpu_sc as plsc

N = 10000
E = 320000
D_IN = 128
D_H = 16
D_OUT = 40

NC, NS = 2, 16
NW = NC * NS
BLK = 128
BPT = 80
EP = NW * BPT * BLK
CHUNK = 16
NPAD = 10240
HR = NPAD // 16
STRIPE = NPAD // NS


def _sc_deg_body(col_hbm, ew_hbm, out_hbm, colv, ewv, hist):
  cid = lax.axis_index("c")
  sid = lax.axis_index("s")
  wid = cid * NS + sid
  pltpu.sync_copy(col_hbm.at[pl.ds(wid * BPT, BPT)], colv)
  pltpu.sync_copy(ew_hbm.at[pl.ds(wid * BPT, BPT)], ewv)

  zero = jnp.zeros((16,), jnp.float32)

  def z(i, carry):
    hist[pl.ds(i * 16, 16)] = zero
    return carry

  lax.fori_loop(0, HR, z, 0)

  def grp(i, carry):
    b = i >> 3
    g = (i & 7) * 16
    idx = colv[b, pl.ds(g, 16)]
    w = ewv[b, pl.ds(g, 16)]
    plsc.addupdate_scatter(hist, [idx], w)
    return carry

  lax.fori_loop(0, BPT * 8, grp, 0)
  pltpu.sync_copy(hist, out_hbm.at[wid])


def _sc_agg_body(g_hbm, row_hbm, col_hbm, ew_hbm, out_hbm,
                 rowv, colv, ewv, rows0, rows1, acc, semg, sems):
  cid = lax.axis_index("c")
  sid = lax.axis_index("s")
  wid = cid * NS + sid
  bufs = (rows0, rows1)
  nch = BPT // CHUNK
  pltpu.sync_copy(row_hbm.at[pl.ds(wid * BPT, BPT)], rowv)
  pltpu.sync_copy(col_hbm.at[pl.ds(wid * BPT, BPT)], colv)
  pltpu.sync_copy(ew_hbm.at[pl.ds(wid * BPT, BPT)], ewv)

  zero = jnp.zeros((16,), jnp.float32)

  def z(i, carry):
    rows0[i, :] = zero
    return carry

  lax.fori_loop(0, STRIPE, z, 0)
  pltpu.sync_copy(rows0.at[pl.ds(0, STRIPE)], acc.at[pl.ds(sid * STRIPE, STRIPE)])
  plsc.subcore_barrier()

  def fire_gathers(c):
    buf = bufs[c % 2]
    ds = []
    for b in range(CHUNK):
      ds.append(pltpu.async_copy(g_hbm.at[rowv.at[c * CHUNK + b]],
                                 buf.at[pl.ds(b * BLK, BLK)], semg))
    return ds

  def fire_scatters(c):
    buf = bufs[c % 2]
    ds = []
    for b in range(CHUNK):
      ds.append(pltpu.async_copy(buf.at[pl.ds(b * BLK, BLK)],
                                 acc.at[colv.at[c * CHUNK + b]], sems,
                                 add=True))
    return ds

  gd = fire_gathers(0)
  sd = [None] * nch
  for c in range(nch):
    for d in gd:
      d.wait()
    if c + 1 < nch:
      if c >= 1:
        for d in sd[c - 1]:
          d.wait()
      gd = fire_gathers(c + 1)

    buf = bufs[c % 2]

    def scale(g, carry):
      blk = g >> 3
      off = (g & 7) * 16
      w16 = ewv[c * CHUNK + blk, pl.ds(off, 16)]
      base = blk * BLK + off
      for l in range(16):
        buf[base + l, :] = buf[base + l, :] * w16[l]
      return carry

    lax.fori_loop(0, CHUNK * 8, scale, 0)
    sd[c] = fire_scatters(c)

  for d in sd[nch - 2]:
    d.wait()
  for d in sd[nch - 1]:
    d.wait()

  plsc.subcore_barrier()
  pltpu.sync_copy(acc.at[pl.ds(sid * STRIPE, STRIPE)],
                  out_hbm.at[cid, pl.ds(sid * STRIPE, STRIPE)])


def _sc_calls():
  mesh = plsc.VectorSubcoreMesh(
      core_axis_name="c", subcore_axis_name="s",
      num_cores=NC, num_subcores=NS)
  deg = pl.kernel(
      _sc_deg_body,
      out_type=jax.ShapeDtypeStruct((NW, NPAD), jnp.float32),
      mesh=mesh,
      scratch_types=[
          pltpu.VMEM((BPT, BLK), jnp.int32),
          pltpu.VMEM((BPT, BLK), jnp.float32),
          pltpu.VMEM((NPAD,), jnp.float32),
      ],
      compiler_params=pltpu.CompilerParams(needs_layout_passes=False),
  )
  agg = pl.kernel(
      _sc_agg_body,
      out_type=jax.ShapeDtypeStruct((NC, NPAD, 16), jnp.float32),
      mesh=mesh,
      scratch_types=[
          pltpu.VMEM((BPT, BLK), jnp.int32),
          pltpu.VMEM((BPT, BLK), jnp.int32),
          pltpu.VMEM((BPT, BLK), jnp.float32),
          pltpu.VMEM((CHUNK * BLK, 16), jnp.float32),
          pltpu.VMEM((CHUNK * BLK, 16), jnp.float32),
          pltpu.VMEM_SHARED((NPAD, 16), jnp.float32),
          pltpu.SemaphoreType.DMA,
          pltpu.SemaphoreType.DMA,
      ],
      compiler_params=pltpu.CompilerParams(
          needs_layout_passes=False, use_tc_tiling_on_sc=False),
  )
  return deg, agg


RB = 2000


def _mm1_body(x_ref, w_ref, o_ref):
  o_ref[...] = jnp.dot(x_ref[...], w_ref[...],
                       preferred_element_type=jnp.float32)


def _dis_body(hist_ref, o_ref):
  deg = jnp.sum(hist_ref[...], axis=0) + 1.0
  o_ref[...] = lax.rsqrt(deg)


def _scale_body(h_ref, d_ref, g_ref, s_ref):
  d = d_ref[...]
  h = h_ref[...]
  g_ref[...] = h * d
  s_ref[...] = h * (d * d)


def _mid_body(p_ref, s_ref, d_ref, b_ref, g_ref, so_ref):
  d = d_ref[...]
  o1 = jnp.maximum((p_ref[0] + p_ref[1]) * d + s_ref[...] + b_ref[...], 0.0)
  g_ref[...] = o1 * d
  so_ref[...] = o1 * (d * d)


def _fin_body(p_ref, so_ref, d_ref, w_ref, b_ref, o_ref):
  agg = (p_ref[0] + p_ref[1]) * d_ref[...] + so_ref[...]
  z = jnp.dot(agg, w_ref[...], preferred_element_type=jnp.float32) + b_ref[...]
  m = jnp.max(z, axis=1, keepdims=True)
  zz = z - m
  lse = jnp.log(jnp.sum(jnp.exp(zz), axis=1, keepdims=True))
  o_ref[...] = zz - lse


def _row_spec(width):
  return pl.BlockSpec((RB, width), lambda i: (i, 0))


def _part_spec():
  return pl.BlockSpec((NC, RB, 16), lambda i: (0, i, 0))


def _full_spec(shape):
  nd = len(shape)
  return pl.BlockSpec(shape, lambda i: (0,) * nd)


def kernel(x, edge_index, edge_weight, W1, b1, W2, b2):
  row = edge_index[0].astype(jnp.int32)
  col = edge_index[1].astype(jnp.int32)
  ew = edge_weight.astype(jnp.float32)
  pad = EP - E
  row2d = jnp.concatenate([row, jnp.zeros((pad,), jnp.int32)]).reshape(NW * BPT, BLK)
  col2d = jnp.concatenate([col, jnp.zeros((pad,), jnp.int32)]).reshape(NW * BPT, BLK)
  ew2d = jnp.concatenate([ew, jnp.zeros((pad,), jnp.float32)]).reshape(NW * BPT, BLK)

  deg_call, agg_call = _sc_calls()

  h = pl.pallas_call(
      _mm1_body,
      grid=(N // RB,),
      in_specs=[pl.BlockSpec((RB, D_IN), lambda i: (i, 0)),
                _full_spec((D_IN, D_H))],
      out_specs=_row_spec(D_H),
      out_shape=jax.ShapeDtypeStruct((N, D_H), jnp.float32),
  )(x, W1)

  hists = deg_call(col2d, ew2d)

  dis_flat = pl.pallas_call(
      _dis_body,
      out_shape=jax.ShapeDtypeStruct((NPAD,), jnp.float32),
  )(hists)
  dis_col = dis_flat.reshape(NPAD, 1)[:N]

  g1, selfh = pl.pallas_call(
      _scale_body,
      grid=(N // RB,),
      in_specs=[_row_spec(D_H), _row_spec(1)],
      out_specs=[_row_spec(D_H), _row_spec(D_H)],
      out_shape=[jax.ShapeDtypeStruct((N, D_H), jnp.float32),
                 jax.ShapeDtypeStruct((N, D_H), jnp.float32)],
  )(h, dis_col)

  parts1 = agg_call(g1, row2d, col2d, ew2d)

  g2, selfo = pl.pallas_call(
      _mid_body,
      grid=(N // RB,),
      in_specs=[_part_spec(), _row_spec(D_H), _row_spec(1),
                _full_spec((1, D_H))],
      out_specs=[_row_spec(D_H), _row_spec(D_H)],
      out_shape=[jax.ShapeDtypeStruct((N, D_H), jnp.float32),
                 jax.ShapeDtypeStruct((N, D_H), jnp.float32)],
  )(parts1, selfh, dis_col, b1.reshape(1, D_H))

  parts2 = agg_call(g2, row2d, col2d, ew2d)

  out = pl.pallas_call(
      _fin_body,
      grid=(N // RB,),
      in_specs=[_part_spec(), _row_spec(D_H), _row_spec(1),
                _full_spec((D_H, D_OUT)), _full_spec((1, D_OUT))],
      out_specs=_row_spec(D_OUT),
      out_shape=jax.ShapeDtypeStruct((N, D_OUT), jnp.float32),
  )(parts2, selfo, dis_col, W2, b2.reshape(1, D_OUT))

  return out
